# Initial kernel scaffold; baseline (speedup 1.0000x reference)
import jax
import jax.numpy as jnp
from jax.experimental import pallas as pl


def kernel(x, vertex_idx, hedge_idx, W0, b0, W1, b1, W2, b2):
    raise NotImplementedError("write your pallas kernel here")


# speedup vs baseline: 4.9255x; 4.9255x over previous
import jax
import jax.numpy as jnp
from jax import lax
from jax.experimental import pallas as pl
from jax.experimental.pallas import tpu as pltpu
from jax.experimental.pallas import tpu_sc as plsc

N = 10000
M = 10000
E = 320000
D = 128
DH = D // 2
NC = 2
NS = 16
L = 16

NP = 10240
MP = 10240
MT = MP // NS
RW = 128
EP = 327680
TT = EP // (RW * NS)
GRP = 4
NGI = TT // GRP
BR = 2048
TB = 128
NPASS = MT // TB


def _sc_agg(xp_flat, vi1, he1, recip):
    compute_deg = recip is None
    mesh = plsc.VectorSubcoreMesh(
        core_axis_name="c", subcore_axis_name="s", num_cores=NC)

    out_type = [jax.ShapeDtypeStruct((NC, NP, DH), jnp.float32)]
    if compute_deg:
        out_type.append(jax.ShapeDtypeStruct((MP,), jnp.float32))

    scratch = (
        [pltpu.VMEM((RW,), jnp.int32) for _ in range(GRP)]
        + [pltpu.VMEM((RW,), jnp.int32) for _ in range(GRP)]
        + [
            pltpu.VMEM((GRP * RW, DH), jnp.float32),
            pltpu.VMEM((TB, DH), jnp.float32),
            pltpu.VMEM((MT,), jnp.float32),
            pltpu.VMEM_SHARED((MP, DH), jnp.float32),
            pltpu.VMEM_SHARED((NP, DH), jnp.float32),
            pltpu.SemaphoreType.DMA,
        ])
    if compute_deg:
        scratch += [
            pltpu.VMEM((RW,), jnp.float32),
            pltpu.VMEM_SHARED((MP,), jnp.float32),
        ]

    def body(xp_ref, vi_ref, he_ref, *rest):
        if compute_deg:
            xa_out, recip_out = rest[0], rest[1]
            rest = rest[2:]
            recip_ref = None
        else:
            recip_ref, xa_out = rest[0], rest[1]
            rest = rest[2:]
        vibufs = rest[:GRP]
        hebufs = rest[GRP:2 * GRP]
        if compute_deg:
            (gbuf, tbuf, rbuf, ys, xa, sem, onesbuf, cnt) = rest[2 * GRP:]
        else:
            (gbuf, tbuf, rbuf, ys, xa, sem) = rest[2 * GRP:]

        c = lax.axis_index("c")
        s = lax.axis_index("s")
        cN = (c * NP).astype(jnp.int32)
        zeros16 = jnp.zeros((L,), jnp.float32)
        ones16 = jnp.ones((L,), jnp.float32)

        @pl.loop(0, TB)
        def _(r):
            for v in range(DH // L):
                tbuf[r, pl.ds(v * L, L)] = zeros16

        for p in range(NPASS):
            pltpu.sync_copy(tbuf, ys.at[pl.ds(s * MT + p * TB, TB)])
            pltpu.sync_copy(tbuf, xa.at[pl.ds(s * MT + p * TB, TB)])
        if compute_deg:
            @pl.loop(0, MT // L)
            def _(r):
                rbuf[pl.ds(r * L, L)] = zeros16
            for v in range(RW // L):
                onesbuf[pl.ds(v * L, L)] = ones16
            pltpu.sync_copy(rbuf, cnt.at[pl.ds(s * MT, MT)])
        plsc.subcore_barrier()

        base = s * TT

        @pl.loop(0, NGI)
        def _(g):
            t0 = (base + g * GRP) * RW
            for q in range(GRP):
                pltpu.sync_copy(vi_ref.at[pl.ds(t0 + q * RW, RW)], vibufs[q])
                pltpu.sync_copy(he_ref.at[pl.ds(t0 + q * RW, RW)], hebufs[q])
            for q in range(GRP):
                for v in range(RW // L):
                    vibufs[q][pl.ds(v * L, L)] = (
                        vibufs[q][pl.ds(v * L, L)] + cN)
            cps = [pltpu.async_copy(xp_ref.at[vibufs[q]],
                                    gbuf.at[pl.ds(q * RW, RW)], sem)
                   for q in range(GRP)]
            if compute_deg:
                for q in range(GRP):
                    pltpu.sync_copy(onesbuf, cnt.at[hebufs[q]], add=True)
            for cp in cps:
                cp.wait()
            cps = [pltpu.async_copy(gbuf.at[pl.ds(q * RW, RW)],
                                    ys.at[hebufs[q]], sem, add=True)
                   for q in range(GRP)]
            for cp in cps:
                cp.wait()

        plsc.subcore_barrier()

        col0 = s * MT
        if compute_deg:
            pltpu.sync_copy(cnt.at[pl.ds(col0, MT)], rbuf)

            @pl.loop(0, MT // L)
            def _(v):
                dv = rbuf[pl.ds(v * L, L)]
                rbuf[pl.ds(v * L, L)] = 1.0 / jnp.maximum(dv, 1.0)

            @pl.when(c == 0)
            def _():
                pltpu.sync_copy(rbuf, recip_out.at[pl.ds(col0, MT)])
        else:
            pltpu.sync_copy(recip_ref.at[pl.ds(col0, MT)], rbuf)

        for p in range(NPASS):
            pltpu.sync_copy(ys.at[pl.ds(col0 + p * TB, TB)], tbuf)

            @pl.loop(0, TB // L)
            def _(k):
                rv16 = rbuf[pl.ds(p * TB + k * L, L)]
                for i in range(L):
                    rv = jnp.broadcast_to(rv16[i], (L,))
                    for v in range(DH // L):
                        tbuf[k * L + i, pl.ds(v * L, L)] = (
                            tbuf[k * L + i, pl.ds(v * L, L)] * rv)

            pltpu.sync_copy(tbuf, ys.at[pl.ds(col0 + p * TB, TB)])
        plsc.subcore_barrier()

        @pl.loop(0, NGI)
        def _(g):
            t0 = (base + g * GRP) * RW
            for q in range(GRP):
                pltpu.sync_copy(vi_ref.at[pl.ds(t0 + q * RW, RW)], vibufs[q])
                pltpu.sync_copy(he_ref.at[pl.ds(t0 + q * RW, RW)], hebufs[q])
            cps = [pltpu.async_copy(ys.at[hebufs[q]],
                                    gbuf.at[pl.ds(q * RW, RW)], sem)
                   for q in range(GRP)]
            for cp in cps:
                cp.wait()
            cps = [pltpu.async_copy(gbuf.at[pl.ds(q * RW, RW)],
                                    xa.at[vibufs[q]], sem, add=True)
                   for q in range(GRP)]
            for cp in cps:
                cp.wait()

        plsc.subcore_barrier()

        for p in range(NPASS):
            pltpu.sync_copy(xa.at[pl.ds(s * MT + p * TB, TB)], tbuf)
            pltpu.sync_copy(tbuf, xa_out.at[c, pl.ds(s * MT + p * TB, TB)])

    run = pl.kernel(body, out_type=tuple(out_type), mesh=mesh,
                    scratch_types=scratch,
                    compiler_params=pltpu.CompilerParams(
                        use_tc_tiling_on_sc=False))
    if compute_deg:
        return run(xp_flat, vi1, he1)
    return run(xp_flat, vi1, he1, recip)[0]


def _tc_first(x, W, b):
    def body(x_ref, w_ref, b_ref, o_ref):
        h = jnp.dot(x_ref[...], w_ref[...],
                    preferred_element_type=jnp.float32) + b_ref[...]
        o_ref[0] = h[:, :DH]
        o_ref[1] = h[:, DH:]

    return pl.pallas_call(
        body,
        grid=(NP // BR,),
        in_specs=[pl.BlockSpec((BR, D), lambda i: (i, 0)),
                  pl.BlockSpec((D, D), lambda i: (0, 0)),
                  pl.BlockSpec((1, D), lambda i: (0, 0))],
        out_specs=pl.BlockSpec((NC, BR, DH), lambda i: (0, i, 0)),
        out_shape=jax.ShapeDtypeStruct((NC, NP, DH), jnp.float32),
    )(x, W, b.reshape(1, D))


def _tc_mid(xp, xa, W, b):
    def body(xp_ref, xa_ref, w_ref, b_ref, o_ref):
        h0 = jnp.maximum(xp_ref[0] + xa_ref[0], 0.0)
        h1 = jnp.maximum(xp_ref[1] + xa_ref[1], 0.0)
        hcat = jnp.concatenate([h0, h1], axis=1)
        h = jnp.dot(hcat, w_ref[...],
                    preferred_element_type=jnp.float32) + b_ref[...]
        o_ref[0] = h[:, :DH]
        o_ref[1] = h[:, DH:]

    return pl.pallas_call(
        body,
        grid=(NP // BR,),
        in_specs=[pl.BlockSpec((NC, BR, DH), lambda i: (0, i, 0)),
                  pl.BlockSpec((NC, BR, DH), lambda i: (0, i, 0)),
                  pl.BlockSpec((D, D), lambda i: (0, 0)),
                  pl.BlockSpec((1, D), lambda i: (0, 0))],
        out_specs=pl.BlockSpec((NC, BR, DH), lambda i: (0, i, 0)),
        out_shape=jax.ShapeDtypeStruct((NC, NP, DH), jnp.float32),
    )(xp, xa, W, b.reshape(1, D))


def _tc_final(xp, xa):
    def body(xp_ref, xa_ref, o_ref):
        h0 = jnp.maximum(xp_ref[0] + xa_ref[0], 0.0)
        h1 = jnp.maximum(xp_ref[1] + xa_ref[1], 0.0)
        o_ref[...] = jnp.concatenate([h0, h1], axis=1)

    return pl.pallas_call(
        body,
        grid=(NP // BR,),
        in_specs=[pl.BlockSpec((NC, BR, DH), lambda i: (0, i, 0)),
                  pl.BlockSpec((NC, BR, DH), lambda i: (0, i, 0))],
        out_specs=pl.BlockSpec((BR, D), lambda i: (i, 0)),
        out_shape=jax.ShapeDtypeStruct((NP, D), jnp.float32),
    )(xp, xa)


def kernel(x, vertex_idx, hedge_idx, W0, b0, W1, b1, W2, b2):
    pad = EP - E
    vpad = N + (jnp.arange(pad, dtype=jnp.int32) % (NP - N))
    hpad = M + (jnp.arange(pad, dtype=jnp.int32) % (MP - M))
    vi1 = jnp.concatenate([vertex_idx.astype(jnp.int32), vpad])
    he1 = jnp.concatenate([hedge_idx.astype(jnp.int32), hpad])
    xpad = jnp.pad(x, ((0, NP - N), (0, 0)))

    xp0 = _tc_first(xpad, W0, b0)
    xa0, recip = _sc_agg(xp0.reshape(NC * NP, DH), vi1, he1, None)
    xp1 = _tc_mid(xp0, xa0, W1, b1)
    xa1 = _sc_agg(xp1.reshape(NC * NP, DH), vi1, he1, recip)
    xp2 = _tc_mid(xp1, xa1, W2, b2)
    xa2 = _sc_agg(xp2.reshape(NC * NP, DH), vi1, he1, recip)
    return _tc_final(xp2, xa2)[:N]


# speedup vs baseline: 7.6058x; 1.5441x over previous
import jax
import jax.numpy as jnp
from jax import lax
from jax.experimental import pallas as pl
from jax.experimental.pallas import tpu as pltpu
from jax.experimental.pallas import tpu_sc as plsc

N = 10000
M = 10000
E = 320000
D = 128
DH = D // 2
NC = 2
NS = 16
L = 16

NP = 10240
MP = 10240
MT = MP // NS
RW = 512
EP = 327680
TT = EP // (RW * NS)
GRP = 1
NGI = TT // GRP
BR = 2048
TB = 128
NPASS = MT // TB


def _sc_agg(xp_flat, vi1, he1, recip):
    compute_deg = recip is None
    mesh = plsc.VectorSubcoreMesh(
        core_axis_name="c", subcore_axis_name="s", num_cores=NC)

    out_type = [jax.ShapeDtypeStruct((NC, NP, DH), jnp.float32)]
    if compute_deg:
        out_type.append(jax.ShapeDtypeStruct((MP,), jnp.float32))

    scratch = (
        [pltpu.VMEM((RW,), jnp.int32) for _ in range(GRP)]
        + [pltpu.VMEM((RW,), jnp.int32) for _ in range(GRP)]
        + [
            pltpu.VMEM((GRP * RW, DH), jnp.float32),
            pltpu.VMEM((TB, DH), jnp.float32),
            pltpu.VMEM((MT,), jnp.float32),
            pltpu.VMEM_SHARED((MP, DH), jnp.float32),
            pltpu.VMEM_SHARED((NP, DH), jnp.float32),
            pltpu.SemaphoreType.DMA,
        ])
    if compute_deg:
        scratch += [
            pltpu.VMEM((RW,), jnp.float32),
            pltpu.VMEM_SHARED((MP,), jnp.float32),
        ]

    def body(xp_ref, vi_ref, he_ref, *rest):
        if compute_deg:
            xa_out, recip_out = rest[0], rest[1]
            rest = rest[2:]
            recip_ref = None
        else:
            recip_ref, xa_out = rest[0], rest[1]
            rest = rest[2:]
        vibufs = rest[:GRP]
        hebufs = rest[GRP:2 * GRP]
        if compute_deg:
            (gbuf, tbuf, rbuf, ys, xa, sem, onesbuf, cnt) = rest[2 * GRP:]
        else:
            (gbuf, tbuf, rbuf, ys, xa, sem) = rest[2 * GRP:]

        c = lax.axis_index("c")
        s = lax.axis_index("s")
        cN = (c * NP).astype(jnp.int32)
        zeros16 = jnp.zeros((L,), jnp.float32)
        ones16 = jnp.ones((L,), jnp.float32)

        @pl.loop(0, TB)
        def _(r):
            for v in range(DH // L):
                tbuf[r, pl.ds(v * L, L)] = zeros16

        for p in range(NPASS):
            pltpu.sync_copy(tbuf, ys.at[pl.ds(s * MT + p * TB, TB)])
            pltpu.sync_copy(tbuf, xa.at[pl.ds(s * MT + p * TB, TB)])
        if compute_deg:
            @pl.loop(0, MT // L)
            def _(r):
                rbuf[pl.ds(r * L, L)] = zeros16
            for v in range(RW // L):
                onesbuf[pl.ds(v * L, L)] = ones16
            pltpu.sync_copy(rbuf, cnt.at[pl.ds(s * MT, MT)])
        plsc.subcore_barrier()

        base = s * TT

        @pl.loop(0, NGI)
        def _(g):
            t0 = (base + g * GRP) * RW
            for q in range(GRP):
                pltpu.sync_copy(vi_ref.at[pl.ds(t0 + q * RW, RW)], vibufs[q])
                pltpu.sync_copy(he_ref.at[pl.ds(t0 + q * RW, RW)], hebufs[q])
            for q in range(GRP):
                for v in range(RW // L):
                    vibufs[q][pl.ds(v * L, L)] = (
                        vibufs[q][pl.ds(v * L, L)] + cN)
            cps = [pltpu.async_copy(xp_ref.at[vibufs[q]],
                                    gbuf.at[pl.ds(q * RW, RW)], sem)
                   for q in range(GRP)]
            if compute_deg:
                for q in range(GRP):
                    pltpu.sync_copy(onesbuf, cnt.at[hebufs[q]], add=True)
            for cp in cps:
                cp.wait()
            cps = [pltpu.async_copy(gbuf.at[pl.ds(q * RW, RW)],
                                    ys.at[hebufs[q]], sem, add=True)
                   for q in range(GRP)]
            for cp in cps:
                cp.wait()

        plsc.subcore_barrier()

        col0 = s * MT
        if compute_deg:
            pltpu.sync_copy(cnt.at[pl.ds(col0, MT)], rbuf)

            @pl.loop(0, MT // L)
            def _(v):
                dv = rbuf[pl.ds(v * L, L)]
                rbuf[pl.ds(v * L, L)] = 1.0 / jnp.maximum(dv, 1.0)

            @pl.when(c == 0)
            def _():
                pltpu.sync_copy(rbuf, recip_out.at[pl.ds(col0, MT)])
        else:
            pltpu.sync_copy(recip_ref.at[pl.ds(col0, MT)], rbuf)

        for p in range(NPASS):
            pltpu.sync_copy(ys.at[pl.ds(col0 + p * TB, TB)], tbuf)

            @pl.loop(0, TB // L)
            def _(k):
                rv16 = rbuf[pl.ds(p * TB + k * L, L)]
                for i in range(L):
                    rv = jnp.broadcast_to(rv16[i], (L,))
                    for v in range(DH // L):
                        tbuf[k * L + i, pl.ds(v * L, L)] = (
                            tbuf[k * L + i, pl.ds(v * L, L)] * rv)

            pltpu.sync_copy(tbuf, ys.at[pl.ds(col0 + p * TB, TB)])
        plsc.subcore_barrier()

        @pl.loop(0, NGI)
        def _(g):
            t0 = (base + g * GRP) * RW
            for q in range(GRP):
                pltpu.sync_copy(vi_ref.at[pl.ds(t0 + q * RW, RW)], vibufs[q])
                pltpu.sync_copy(he_ref.at[pl.ds(t0 + q * RW, RW)], hebufs[q])
            cps = [pltpu.async_copy(ys.at[hebufs[q]],
                                    gbuf.at[pl.ds(q * RW, RW)], sem)
                   for q in range(GRP)]
            for cp in cps:
                cp.wait()
            cps = [pltpu.async_copy(gbuf.at[pl.ds(q * RW, RW)],
                                    xa.at[vibufs[q]], sem, add=True)
                   for q in range(GRP)]
            for cp in cps:
                cp.wait()

        plsc.subcore_barrier()

        for p in range(NPASS):
            pltpu.sync_copy(xa.at[pl.ds(s * MT + p * TB, TB)], tbuf)
            pltpu.sync_copy(tbuf, xa_out.at[c, pl.ds(s * MT + p * TB, TB)])

    run = pl.kernel(body, out_type=tuple(out_type), mesh=mesh,
                    scratch_types=scratch,
                    compiler_params=pltpu.CompilerParams(
                        use_tc_tiling_on_sc=False))
    if compute_deg:
        return run(xp_flat, vi1, he1)
    return run(xp_flat, vi1, he1, recip)[0]


def _tc_first(x, W, b):
    def body(x_ref, w_ref, b_ref, o_ref):
        h = jnp.dot(x_ref[...], w_ref[...],
                    preferred_element_type=jnp.float32) + b_ref[...]
        o_ref[0] = h[:, :DH]
        o_ref[1] = h[:, DH:]

    return pl.pallas_call(
        body,
        grid=(NP // BR,),
        in_specs=[pl.BlockSpec((BR, D), lambda i: (i, 0)),
                  pl.BlockSpec((D, D), lambda i: (0, 0)),
                  pl.BlockSpec((1, D), lambda i: (0, 0))],
        out_specs=pl.BlockSpec((NC, BR, DH), lambda i: (0, i, 0)),
        out_shape=jax.ShapeDtypeStruct((NC, NP, DH), jnp.float32),
    )(x, W, b.reshape(1, D))


def _tc_mid(xp, xa, W, b):
    def body(xp_ref, xa_ref, w_ref, b_ref, o_ref):
        h0 = jnp.maximum(xp_ref[0] + xa_ref[0], 0.0)
        h1 = jnp.maximum(xp_ref[1] + xa_ref[1], 0.0)
        hcat = jnp.concatenate([h0, h1], axis=1)
        h = jnp.dot(hcat, w_ref[...],
                    preferred_element_type=jnp.float32) + b_ref[...]
        o_ref[0] = h[:, :DH]
        o_ref[1] = h[:, DH:]

    return pl.pallas_call(
        body,
        grid=(NP // BR,),
        in_specs=[pl.BlockSpec((NC, BR, DH), lambda i: (0, i, 0)),
                  pl.BlockSpec((NC, BR, DH), lambda i: (0, i, 0)),
                  pl.BlockSpec((D, D), lambda i: (0, 0)),
                  pl.BlockSpec((1, D), lambda i: (0, 0))],
        out_specs=pl.BlockSpec((NC, BR, DH), lambda i: (0, i, 0)),
        out_shape=jax.ShapeDtypeStruct((NC, NP, DH), jnp.float32),
    )(xp, xa, W, b.reshape(1, D))


def _tc_final(xp, xa):
    def body(xp_ref, xa_ref, o_ref):
        h0 = jnp.maximum(xp_ref[0] + xa_ref[0], 0.0)
        h1 = jnp.maximum(xp_ref[1] + xa_ref[1], 0.0)
        o_ref[...] = jnp.concatenate([h0, h1], axis=1)

    return pl.pallas_call(
        body,
        grid=(NP // BR,),
        in_specs=[pl.BlockSpec((NC, BR, DH), lambda i: (0, i, 0)),
                  pl.BlockSpec((NC, BR, DH), lambda i: (0, i, 0))],
        out_specs=pl.BlockSpec((BR, D), lambda i: (i, 0)),
        out_shape=jax.ShapeDtypeStruct((NP, D), jnp.float32),
    )(xp, xa)


def kernel(x, vertex_idx, hedge_idx, W0, b0, W1, b1, W2, b2):
    pad = EP - E
    vpad = N + (jnp.arange(pad, dtype=jnp.int32) % (NP - N))
    hpad = M + (jnp.arange(pad, dtype=jnp.int32) % (MP - M))
    vi1 = jnp.concatenate([vertex_idx.astype(jnp.int32), vpad])
    he1 = jnp.concatenate([hedge_idx.astype(jnp.int32), hpad])
    xpad = jnp.pad(x, ((0, NP - N), (0, 0)))

    xp0 = _tc_first(xpad, W0, b0)
    xa0, recip = _sc_agg(xp0.reshape(NC * NP, DH), vi1, he1, None)
    xp1 = _tc_mid(xp0, xa0, W1, b1)
    xa1 = _sc_agg(xp1.reshape(NC * NP, DH), vi1, he1, recip)
    xp2 = _tc_mid(xp1, xa1, W2, b2)
    xa2 = _sc_agg(xp2.reshape(NC * NP, DH), vi1, he1, recip)
    return _tc_final(xp2, xa2)[:N]


# speedup vs baseline: 8.7721x; 1.1534x over previous
import jax
import jax.numpy as jnp
from jax import lax
from jax.experimental import pallas as pl
from jax.experimental.pallas import tpu as pltpu
from jax.experimental.pallas import tpu_sc as plsc

N = 10000
M = 10000
E = 320000
D = 128
DH = D // 2
NC = 2
NS = 16
L = 16

NP = 10240
MP = 10240
MT = MP // NS
RW = 256
EP = 327680
NGI = EP // (RW * NS)
BR = 2048
TB = 128
NPASS = MT // TB


def _sc_agg(xp_flat, vi1, he1, recip):
    compute_deg = recip is None
    mesh = plsc.VectorSubcoreMesh(
        core_axis_name="c", subcore_axis_name="s", num_cores=NC)

    out_type = [jax.ShapeDtypeStruct((NC, NP, DH), jnp.float32)]
    if compute_deg:
        out_type.append(jax.ShapeDtypeStruct((MP,), jnp.float32))

    scratch = (
        [pltpu.VMEM((RW,), jnp.int32) for _ in range(4)]
        + [
            pltpu.VMEM((RW, DH), jnp.float32),
            pltpu.VMEM((RW, DH), jnp.float32),
            pltpu.VMEM((TB, DH), jnp.float32),
            pltpu.VMEM((MT,), jnp.float32),
            pltpu.VMEM_SHARED((MP, DH), jnp.float32),
            pltpu.VMEM_SHARED((NP, DH), jnp.float32),
        ]
        + [pltpu.SemaphoreType.DMA for _ in range(7)]
    )
    if compute_deg:
        scratch += [
            pltpu.VMEM((RW,), jnp.float32),
            pltpu.VMEM_SHARED((MP,), jnp.float32),
        ]

    def body(xp_ref, vi_ref, he_ref, *rest):
        if compute_deg:
            xa_out, recip_out = rest[0], rest[1]
            rest = rest[2:]
            recip_ref = None
        else:
            recip_ref, xa_out = rest[0], rest[1]
            rest = rest[2:]
        (viA, viB, heA, heB, gbA, gbB, tbuf, rbuf, ys, xa,
         gsemA, gsemB, ssemA, ssemB, isemA, isemB, sem) = rest[:17]
        if compute_deg:
            onesbuf, cnt = rest[17], rest[18]

        c = lax.axis_index("c")
        s = lax.axis_index("s")
        cN = (c * NP).astype(jnp.int32)
        zeros16 = jnp.zeros((L,), jnp.float32)
        ones16 = jnp.ones((L,), jnp.float32)

        @pl.loop(0, TB)
        def _(r):
            for v in range(DH // L):
                tbuf[r, pl.ds(v * L, L)] = zeros16

        for p in range(NPASS):
            pltpu.sync_copy(tbuf, ys.at[pl.ds(s * MT + p * TB, TB)])
            pltpu.sync_copy(tbuf, xa.at[pl.ds(s * MT + p * TB, TB)])
        if compute_deg:
            @pl.loop(0, MT // L)
            def _(r):
                rbuf[pl.ds(r * L, L)] = zeros16
            for v in range(RW // L):
                onesbuf[pl.ds(v * L, L)] = ones16
            pltpu.sync_copy(rbuf, cnt.at[pl.ds(s * MT, MT)])
        plsc.subcore_barrier()

        base = s * NGI

        bufsA = (viA, heA, gbA, gsemA, ssemA, isemA)
        bufsB = (viB, heB, gbB, gsemB, ssemB, isemB)

        def _run_phase(gsrc, sdst, adjust, hist):
            def idx_launch(g, P):
                vib, heb, _, _, _, isem = P
                t0 = (base + g) * RW
                cpv = pltpu.async_copy(vi_ref.at[pl.ds(t0, RW)], vib, isem)
                cph = pltpu.async_copy(he_ref.at[pl.ds(t0, RW)], heb, isem)
                cpv.wait()
                cph.wait()
                if adjust:
                    @pl.loop(0, RW // L)
                    def _(v):
                        vib[pl.ds(v * L, L)] = vib[pl.ds(v * L, L)] + cN

            def gather_fire(P):
                vib, heb, gb, gsem, _, _ = P
                pltpu.async_copy(gsrc(vib, heb), gb, gsem)

            def gather_wait(P):
                vib, heb, gb, gsem, _, _ = P
                pltpu.make_async_copy(gsrc(vib, heb), gb, gsem).wait()

            def scatter_fire(P):
                vib, heb, gb, _, ssem, _ = P
                pltpu.async_copy(gb, sdst(vib, heb), ssem, add=True)
                if hist:
                    pltpu.async_copy(onesbuf, cnt.at[heb], ssem, add=True)

            def scatter_wait(P):
                vib, heb, gb, _, ssem, _ = P
                pltpu.make_async_copy(gb, sdst(vib, heb), ssem).wait()
                if hist:
                    pltpu.make_async_copy(onesbuf, cnt.at[heb], ssem).wait()

            idx_launch(0, bufsA)
            gather_fire(bufsA)

            def half(P, Q, g):
                gather_wait(P)
                scatter_fire(P)

                @pl.when(g + 1 < NGI)
                def _():
                    @pl.when(g >= 1)
                    def _():
                        scatter_wait(Q)
                    idx_launch(g + 1, Q)
                    gather_fire(Q)

            @pl.loop(0, NGI)
            def _(g):
                even = (g % 2) == 0

                @pl.when(even)
                def _():
                    half(bufsA, bufsB, g)

                @pl.when(jnp.logical_not(even))
                def _():
                    half(bufsB, bufsA, g)

            scatter_wait(bufsA)
            scatter_wait(bufsB)

        _run_phase(gsrc=lambda vib, heb: xp_ref.at[vib],
                   sdst=lambda vib, heb: ys.at[heb],
                   adjust=True, hist=compute_deg)
        plsc.subcore_barrier()

        col0 = s * MT
        if compute_deg:
            pltpu.sync_copy(cnt.at[pl.ds(col0, MT)], rbuf)

            @pl.loop(0, MT // L)
            def _(v):
                dv = rbuf[pl.ds(v * L, L)]
                rbuf[pl.ds(v * L, L)] = 1.0 / jnp.maximum(dv, 1.0)

            @pl.when(c == 0)
            def _():
                pltpu.sync_copy(rbuf, recip_out.at[pl.ds(col0, MT)])
        else:
            pltpu.sync_copy(recip_ref.at[pl.ds(col0, MT)], rbuf)

        for p in range(NPASS):
            pltpu.sync_copy(ys.at[pl.ds(col0 + p * TB, TB)], tbuf)

            @pl.loop(0, TB // L)
            def _(k):
                rv16 = rbuf[pl.ds(p * TB + k * L, L)]
                for i in range(L):
                    rv = jnp.broadcast_to(rv16[i], (L,))
                    for v in range(DH // L):
                        tbuf[k * L + i, pl.ds(v * L, L)] = (
                            tbuf[k * L + i, pl.ds(v * L, L)] * rv)

            pltpu.sync_copy(tbuf, ys.at[pl.ds(col0 + p * TB, TB)])
        plsc.subcore_barrier()

        _run_phase(gsrc=lambda vib, heb: ys.at[heb],
                   sdst=lambda vib, heb: xa.at[vib],
                   adjust=False, hist=False)
        plsc.subcore_barrier()

        for p in range(NPASS):
            pltpu.sync_copy(xa.at[pl.ds(s * MT + p * TB, TB)], tbuf)
            pltpu.sync_copy(tbuf, xa_out.at[c, pl.ds(s * MT + p * TB, TB)])

    run = pl.kernel(body, out_type=tuple(out_type), mesh=mesh,
                    scratch_types=scratch,
                    compiler_params=pltpu.CompilerParams(
                        use_tc_tiling_on_sc=False))
    if compute_deg:
        return run(xp_flat, vi1, he1)
    return run(xp_flat, vi1, he1, recip)[0]


def _tc_first(x, W, b):
    def body(x_ref, w_ref, b_ref, o_ref):
        h = jnp.dot(x_ref[...], w_ref[...],
                    preferred_element_type=jnp.float32) + b_ref[...]
        o_ref[0] = h[:, :DH]
        o_ref[1] = h[:, DH:]

    return pl.pallas_call(
        body,
        grid=(NP // BR,),
        in_specs=[pl.BlockSpec((BR, D), lambda i: (i, 0)),
                  pl.BlockSpec((D, D), lambda i: (0, 0)),
                  pl.BlockSpec((1, D), lambda i: (0, 0))],
        out_specs=pl.BlockSpec((NC, BR, DH), lambda i: (0, i, 0)),
        out_shape=jax.ShapeDtypeStruct((NC, NP, DH), jnp.float32),
    )(x, W, b.reshape(1, D))


def _tc_mid(xp, xa, W, b):
    def body(xp_ref, xa_ref, w_ref, b_ref, o_ref):
        h0 = jnp.maximum(xp_ref[0] + xa_ref[0], 0.0)
        h1 = jnp.maximum(xp_ref[1] + xa_ref[1], 0.0)
        hcat = jnp.concatenate([h0, h1], axis=1)
        h = jnp.dot(hcat, w_ref[...],
                    preferred_element_type=jnp.float32) + b_ref[...]
        o_ref[0] = h[:, :DH]
        o_ref[1] = h[:, DH:]

    return pl.pallas_call(
        body,
        grid=(NP // BR,),
        in_specs=[pl.BlockSpec((NC, BR, DH), lambda i: (0, i, 0)),
                  pl.BlockSpec((NC, BR, DH), lambda i: (0, i, 0)),
                  pl.BlockSpec((D, D), lambda i: (0, 0)),
                  pl.BlockSpec((1, D), lambda i: (0, 0))],
        out_specs=pl.BlockSpec((NC, BR, DH), lambda i: (0, i, 0)),
        out_shape=jax.ShapeDtypeStruct((NC, NP, DH), jnp.float32),
    )(xp, xa, W, b.reshape(1, D))


def _tc_final(xp, xa):
    def body(xp_ref, xa_ref, o_ref):
        h0 = jnp.maximum(xp_ref[0] + xa_ref[0], 0.0)
        h1 = jnp.maximum(xp_ref[1] + xa_ref[1], 0.0)
        o_ref[...] = jnp.concatenate([h0, h1], axis=1)

    return pl.pallas_call(
        body,
        grid=(NP // BR,),
        in_specs=[pl.BlockSpec((NC, BR, DH), lambda i: (0, i, 0)),
                  pl.BlockSpec((NC, BR, DH), lambda i: (0, i, 0))],
        out_specs=pl.BlockSpec((BR, D), lambda i: (i, 0)),
        out_shape=jax.ShapeDtypeStruct((NP, D), jnp.float32),
    )(xp, xa)


def kernel(x, vertex_idx, hedge_idx, W0, b0, W1, b1, W2, b2):
    pad = EP - E
    vpad = N + (jnp.arange(pad, dtype=jnp.int32) % (NP - N))
    hpad = M + (jnp.arange(pad, dtype=jnp.int32) % (MP - M))
    vi1 = jnp.concatenate([vertex_idx.astype(jnp.int32), vpad])
    he1 = jnp.concatenate([hedge_idx.astype(jnp.int32), hpad])
    xpad = jnp.pad(x, ((0, NP - N), (0, 0)))

    xp0 = _tc_first(xpad, W0, b0)
    xa0, recip = _sc_agg(xp0.reshape(NC * NP, DH), vi1, he1, None)
    xp1 = _tc_mid(xp0, xa0, W1, b1)
    xa1 = _sc_agg(xp1.reshape(NC * NP, DH), vi1, he1, recip)
    xp2 = _tc_mid(xp1, xa1, W2, b2)
    xa2 = _sc_agg(xp2.reshape(NC * NP, DH), vi1, he1, recip)
    return _tc_final(xp2, xa2)[:N]


# speedup vs baseline: 10.6941x; 1.2191x over previous
import jax
import jax.numpy as jnp
from jax import lax
from jax.experimental import pallas as pl
from jax.experimental.pallas import tpu as pltpu
from jax.experimental.pallas import tpu_sc as plsc

N = 10000
M = 10000
E = 320000
D = 128
DH = D // 2
NC = 2
NS = 16
L = 16

NP = 10240
MP = 10240
MT = MP // NS
RW = 160
EP = 327680
NGI = EP // (RW * NS)
NBUF = 3
BR = 2048
TB = 128
NPASS = MT // TB


def _sc_agg(xp_flat, vi1, he1, recip):
    compute_deg = recip is None
    mesh = plsc.VectorSubcoreMesh(
        core_axis_name="c", subcore_axis_name="s", num_cores=NC)

    out_type = [jax.ShapeDtypeStruct((NC, NP, DH), jnp.float32)]
    if compute_deg:
        out_type.append(jax.ShapeDtypeStruct((MP,), jnp.float32))

    scratch = (
        [pltpu.VMEM((RW,), jnp.int32) for _ in range(2 * NBUF)]
        + [pltpu.VMEM((RW, DH), jnp.float32) for _ in range(NBUF)]
        + [
            pltpu.VMEM((TB, DH), jnp.float32),
            pltpu.VMEM((MT,), jnp.float32),
            pltpu.VMEM_SHARED((MP, DH), jnp.float32),
            pltpu.VMEM_SHARED((NP, DH), jnp.float32),
        ]
        + [pltpu.SemaphoreType.DMA for _ in range(3 * NBUF + 1)]
    )
    if compute_deg:
        scratch += [
            pltpu.VMEM((RW,), jnp.float32),
            pltpu.VMEM_SHARED((MP,), jnp.float32),
        ]

    def body(xp_ref, vi_ref, he_ref, *rest):
        if compute_deg:
            xa_out, recip_out = rest[0], rest[1]
            rest = rest[2:]
            recip_ref = None
        else:
            recip_ref, xa_out = rest[0], rest[1]
            rest = rest[2:]
        vibs = rest[:NBUF]
        hebs = rest[NBUF:2 * NBUF]
        gbs = rest[2 * NBUF:3 * NBUF]
        tbuf, rbuf, ys, xa = rest[3 * NBUF:3 * NBUF + 4]
        sems = rest[3 * NBUF + 4:6 * NBUF + 4]
        gsems, ssems, isems = sems[:NBUF], sems[NBUF:2 * NBUF], sems[2 * NBUF:]
        sem = rest[6 * NBUF + 4]
        if compute_deg:
            onesbuf, cnt = rest[6 * NBUF + 5], rest[6 * NBUF + 6]

        c = lax.axis_index("c")
        s = lax.axis_index("s")
        cN = (c * NP).astype(jnp.int32)
        zeros16 = jnp.zeros((L,), jnp.float32)
        ones16 = jnp.ones((L,), jnp.float32)

        @pl.loop(0, TB)
        def _(r):
            for v in range(DH // L):
                tbuf[r, pl.ds(v * L, L)] = zeros16

        zcps = []
        for p in range(NPASS):
            zcps.append(pltpu.async_copy(
                tbuf, ys.at[pl.ds(s * MT + p * TB, TB)], sem))
            zcps.append(pltpu.async_copy(
                tbuf, xa.at[pl.ds(s * MT + p * TB, TB)], sem))
        for cp in zcps:
            cp.wait()
        if compute_deg:
            @pl.loop(0, MT // L)
            def _(r):
                rbuf[pl.ds(r * L, L)] = zeros16
            for v in range(RW // L):
                onesbuf[pl.ds(v * L, L)] = ones16
            pltpu.sync_copy(rbuf, cnt.at[pl.ds(s * MT, MT)])
        plsc.subcore_barrier()

        base = s * NGI

        bufsets = [(vibs[r], hebs[r], gbs[r], gsems[r], ssems[r], isems[r])
                   for r in range(NBUF)]

        def _run_phase(gsrc, sdst, adjust, hist):
            def idx_launch(g, P):
                vib, heb, _, _, _, isem = P
                t0 = (base + g) * RW
                cpv = pltpu.async_copy(vi_ref.at[pl.ds(t0, RW)], vib, isem)
                cph = pltpu.async_copy(he_ref.at[pl.ds(t0, RW)], heb, isem)
                cpv.wait()
                cph.wait()
                if adjust:
                    @pl.loop(0, RW // L)
                    def _(v):
                        vib[pl.ds(v * L, L)] = vib[pl.ds(v * L, L)] + cN

            def gather_fire(P):
                vib, heb, gb, gsem, _, _ = P
                pltpu.async_copy(gsrc(vib, heb), gb, gsem)

            def gather_wait(P):
                vib, heb, gb, gsem, _, _ = P
                pltpu.make_async_copy(gsrc(vib, heb), gb, gsem).wait()

            def scatter_fire(P):
                vib, heb, gb, _, ssem, _ = P
                pltpu.async_copy(gb, sdst(vib, heb), ssem, add=True)
                if hist:
                    pltpu.async_copy(onesbuf, cnt.at[heb], ssem, add=True)

            def scatter_wait(P):
                vib, heb, gb, _, ssem, _ = P
                pltpu.make_async_copy(gb, sdst(vib, heb), ssem).wait()
                if hist:
                    pltpu.make_async_copy(onesbuf, cnt.at[heb], ssem).wait()

            idx_launch(0, bufsets[0])
            gather_fire(bufsets[0])
            idx_launch(1, bufsets[1])
            gather_fire(bufsets[1])

            def step(r, g):
                X = bufsets[r]
                Z = bufsets[(r + 2) % NBUF]
                gather_wait(X)
                scatter_fire(X)

                @pl.when(g + 2 < NGI)
                def _():
                    @pl.when(g >= 1)
                    def _():
                        scatter_wait(Z)
                    idx_launch(g + 2, Z)
                    gather_fire(Z)

            @pl.loop(0, NGI)
            def _(g):
                gm = g % NBUF
                for r in range(NBUF):
                    @pl.when(gm == r)
                    def _(r=r):
                        step(r, g)

            for r in range(NBUF):
                scatter_wait(bufsets[r])

        _run_phase(gsrc=lambda vib, heb: xp_ref.at[vib],
                   sdst=lambda vib, heb: ys.at[heb],
                   adjust=True, hist=compute_deg)
        plsc.subcore_barrier()

        col0 = s * MT
        if compute_deg:
            pltpu.sync_copy(cnt.at[pl.ds(col0, MT)], rbuf)

            @pl.loop(0, MT // L)
            def _(v):
                dv = rbuf[pl.ds(v * L, L)]
                rbuf[pl.ds(v * L, L)] = 1.0 / jnp.maximum(dv, 1.0)

            @pl.when(c == 0)
            def _():
                pltpu.sync_copy(rbuf, recip_out.at[pl.ds(col0, MT)])
        else:
            pltpu.sync_copy(recip_ref.at[pl.ds(col0, MT)], rbuf)

        for p in range(NPASS):
            pltpu.sync_copy(ys.at[pl.ds(col0 + p * TB, TB)], tbuf)

            @pl.loop(0, TB // L)
            def _(k):
                rv16 = rbuf[pl.ds(p * TB + k * L, L)]
                for i in range(L):
                    rv = jnp.broadcast_to(rv16[i], (L,))
                    for v in range(DH // L):
                        tbuf[k * L + i, pl.ds(v * L, L)] = (
                            tbuf[k * L + i, pl.ds(v * L, L)] * rv)

            pltpu.sync_copy(tbuf, ys.at[pl.ds(col0 + p * TB, TB)])
        plsc.subcore_barrier()

        _run_phase(gsrc=lambda vib, heb: ys.at[heb],
                   sdst=lambda vib, heb: xa.at[vib],
                   adjust=False, hist=False)
        plsc.subcore_barrier()

        for p in range(NPASS):
            pltpu.sync_copy(xa.at[pl.ds(s * MT + p * TB, TB)], tbuf)
            pltpu.sync_copy(tbuf, xa_out.at[c, pl.ds(s * MT + p * TB, TB)])

    run = pl.kernel(body, out_type=tuple(out_type), mesh=mesh,
                    scratch_types=scratch,
                    compiler_params=pltpu.CompilerParams(
                        use_tc_tiling_on_sc=False))
    if compute_deg:
        return run(xp_flat, vi1, he1)
    return run(xp_flat, vi1, he1, recip)[0]


def _tc_first(x, W, b):
    def body(x_ref, w_ref, b_ref, o_ref):
        h = jnp.dot(x_ref[...], w_ref[...],
                    preferred_element_type=jnp.float32) + b_ref[...]
        o_ref[0] = h[:, :DH]
        o_ref[1] = h[:, DH:]

    return pl.pallas_call(
        body,
        grid=(NP // BR,),
        in_specs=[pl.BlockSpec((BR, D), lambda i: (i, 0)),
                  pl.BlockSpec((D, D), lambda i: (0, 0)),
                  pl.BlockSpec((1, D), lambda i: (0, 0))],
        out_specs=pl.BlockSpec((NC, BR, DH), lambda i: (0, i, 0)),
        out_shape=jax.ShapeDtypeStruct((NC, NP, DH), jnp.float32),
    )(x, W, b.reshape(1, D))


def _tc_mid(xp, xa, W, b):
    def body(xp_ref, xa_ref, w_ref, b_ref, o_ref):
        h0 = jnp.maximum(xp_ref[0] + xa_ref[0], 0.0)
        h1 = jnp.maximum(xp_ref[1] + xa_ref[1], 0.0)
        hcat = jnp.concatenate([h0, h1], axis=1)
        h = jnp.dot(hcat, w_ref[...],
                    preferred_element_type=jnp.float32) + b_ref[...]
        o_ref[0] = h[:, :DH]
        o_ref[1] = h[:, DH:]

    return pl.pallas_call(
        body,
        grid=(NP // BR,),
        in_specs=[pl.BlockSpec((NC, BR, DH), lambda i: (0, i, 0)),
                  pl.BlockSpec((NC, BR, DH), lambda i: (0, i, 0)),
                  pl.BlockSpec((D, D), lambda i: (0, 0)),
                  pl.BlockSpec((1, D), lambda i: (0, 0))],
        out_specs=pl.BlockSpec((NC, BR, DH), lambda i: (0, i, 0)),
        out_shape=jax.ShapeDtypeStruct((NC, NP, DH), jnp.float32),
    )(xp, xa, W, b.reshape(1, D))


def _tc_final(xp, xa):
    def body(xp_ref, xa_ref, o_ref):
        h0 = jnp.maximum(xp_ref[0] + xa_ref[0], 0.0)
        h1 = jnp.maximum(xp_ref[1] + xa_ref[1], 0.0)
        o_ref[...] = jnp.concatenate([h0, h1], axis=1)

    return pl.pallas_call(
        body,
        grid=(NP // BR,),
        in_specs=[pl.BlockSpec((NC, BR, DH), lambda i: (0, i, 0)),
                  pl.BlockSpec((NC, BR, DH), lambda i: (0, i, 0))],
        out_specs=pl.BlockSpec((BR, D), lambda i: (i, 0)),
        out_shape=jax.ShapeDtypeStruct((NP, D), jnp.float32),
    )(xp, xa)


def kernel(x, vertex_idx, hedge_idx, W0, b0, W1, b1, W2, b2):
    pad = EP - E
    vpad = N + (jnp.arange(pad, dtype=jnp.int32) % (NP - N))
    hpad = M + (jnp.arange(pad, dtype=jnp.int32) % (MP - M))
    vi1 = jnp.concatenate([vertex_idx.astype(jnp.int32), vpad])
    he1 = jnp.concatenate([hedge_idx.astype(jnp.int32), hpad])
    xpad = jnp.pad(x, ((0, NP - N), (0, 0)))

    xp0 = _tc_first(xpad, W0, b0)
    xa0, recip = _sc_agg(xp0.reshape(NC * NP, DH), vi1, he1, None)
    xp1 = _tc_mid(xp0, xa0, W1, b1)
    xa1 = _sc_agg(xp1.reshape(NC * NP, DH), vi1, he1, recip)
    xp2 = _tc_mid(xp1, xa1, W2, b2)
    xa2 = _sc_agg(xp2.reshape(NC * NP, DH), vi1, he1, recip)
    return _tc_final(xp2, xa2)[:N]


# speedup vs baseline: 11.1257x; 1.0404x over previous
import jax
import jax.numpy as jnp
from jax import lax
from jax.experimental import pallas as pl
from jax.experimental.pallas import tpu as pltpu
from jax.experimental.pallas import tpu_sc as plsc

N = 10000
M = 10000
E = 320000
D = 128
DH = D // 2
NC = 2
NS = 16
L = 16

NP = 10240
MP = 10240
MT = MP // NS
RW = 128
EP = 327680
NGI = EP // (RW * NS)
NBUF = 4
BR = 2048
TB = 128
NPASS = MT // TB


def _sc_agg(xp_flat, vi1, he1, recip):
    compute_deg = recip is None
    mesh = plsc.VectorSubcoreMesh(
        core_axis_name="c", subcore_axis_name="s", num_cores=NC)

    out_type = [jax.ShapeDtypeStruct((NC, NP, DH), jnp.float32)]
    if compute_deg:
        out_type.append(jax.ShapeDtypeStruct((MP,), jnp.float32))

    scratch = (
        [pltpu.VMEM((RW,), jnp.int32) for _ in range(2 * NBUF)]
        + [pltpu.VMEM((RW, DH), jnp.float32) for _ in range(NBUF)]
        + [
            pltpu.VMEM((TB, DH), jnp.float32),
            pltpu.VMEM((MT,), jnp.float32),
            pltpu.VMEM_SHARED((MP, DH), jnp.float32),
            pltpu.VMEM_SHARED((NP, DH), jnp.float32),
        ]
        + [pltpu.SemaphoreType.DMA for _ in range(3 * NBUF + 1)]
    )
    if compute_deg:
        scratch += [
            pltpu.VMEM((RW,), jnp.float32),
            pltpu.VMEM_SHARED((MP,), jnp.float32),
        ]

    def body(xp_ref, vi_ref, he_ref, *rest):
        if compute_deg:
            xa_out, recip_out = rest[0], rest[1]
            rest = rest[2:]
            recip_ref = None
        else:
            recip_ref, xa_out = rest[0], rest[1]
            rest = rest[2:]
        vibs = rest[:NBUF]
        hebs = rest[NBUF:2 * NBUF]
        gbs = rest[2 * NBUF:3 * NBUF]
        tbuf, rbuf, ys, xa = rest[3 * NBUF:3 * NBUF + 4]
        sems = rest[3 * NBUF + 4:6 * NBUF + 4]
        gsems, ssems, isems = sems[:NBUF], sems[NBUF:2 * NBUF], sems[2 * NBUF:]
        sem = rest[6 * NBUF + 4]
        if compute_deg:
            onesbuf, cnt = rest[6 * NBUF + 5], rest[6 * NBUF + 6]

        c = lax.axis_index("c")
        s = lax.axis_index("s")
        cN = (c * NP).astype(jnp.int32)
        zeros16 = jnp.zeros((L,), jnp.float32)
        ones16 = jnp.ones((L,), jnp.float32)

        @pl.loop(0, TB)
        def _(r):
            for v in range(DH // L):
                tbuf[r, pl.ds(v * L, L)] = zeros16

        zcps = []
        for p in range(NPASS):
            zcps.append(pltpu.async_copy(
                tbuf, ys.at[pl.ds(s * MT + p * TB, TB)], sem))
            zcps.append(pltpu.async_copy(
                tbuf, xa.at[pl.ds(s * MT + p * TB, TB)], sem))
        for cp in zcps:
            cp.wait()
        if compute_deg:
            @pl.loop(0, MT // L)
            def _(r):
                rbuf[pl.ds(r * L, L)] = zeros16
            for v in range(RW // L):
                onesbuf[pl.ds(v * L, L)] = ones16
            pltpu.sync_copy(rbuf, cnt.at[pl.ds(s * MT, MT)])
        plsc.subcore_barrier()

        base = s * NGI

        bufsets = [(vibs[r], hebs[r], gbs[r], gsems[r], ssems[r], isems[r])
                   for r in range(NBUF)]

        def _run_phase(gsrc, sdst, adjust, hist):
            def idx_launch(g, P):
                vib, heb, _, _, _, isem = P
                t0 = (base + g) * RW
                cpv = pltpu.async_copy(vi_ref.at[pl.ds(t0, RW)], vib, isem)
                cph = pltpu.async_copy(he_ref.at[pl.ds(t0, RW)], heb, isem)
                cpv.wait()
                cph.wait()
                if adjust:
                    @pl.loop(0, RW // L)
                    def _(v):
                        vib[pl.ds(v * L, L)] = vib[pl.ds(v * L, L)] + cN

            def gather_fire(P):
                vib, heb, gb, gsem, _, _ = P
                pltpu.async_copy(gsrc(vib, heb), gb, gsem)

            def gather_wait(P):
                vib, heb, gb, gsem, _, _ = P
                pltpu.make_async_copy(gsrc(vib, heb), gb, gsem).wait()

            def scatter_fire(P):
                vib, heb, gb, _, ssem, _ = P
                pltpu.async_copy(gb, sdst(vib, heb), ssem, add=True)
                if hist:
                    pltpu.async_copy(onesbuf, cnt.at[heb], ssem, add=True)

            def scatter_wait(P):
                vib, heb, gb, _, ssem, _ = P
                pltpu.make_async_copy(gb, sdst(vib, heb), ssem).wait()
                if hist:
                    pltpu.make_async_copy(onesbuf, cnt.at[heb], ssem).wait()

            for r in range(NBUF - 1):
                idx_launch(r, bufsets[r])
                gather_fire(bufsets[r])

            def step(r, g):
                X = bufsets[r]
                Z = bufsets[(r + NBUF - 1) % NBUF]
                gather_wait(X)
                scatter_fire(X)

                @pl.when(g + NBUF - 1 < NGI)
                def _():
                    @pl.when(g >= 1)
                    def _():
                        scatter_wait(Z)
                    idx_launch(g + NBUF - 1, Z)
                    gather_fire(Z)

            @pl.loop(0, NGI)
            def _(g):
                gm = g % NBUF
                for r in range(NBUF):
                    @pl.when(gm == r)
                    def _(r=r):
                        step(r, g)

            for r in range(NBUF):
                scatter_wait(bufsets[r])

        _run_phase(gsrc=lambda vib, heb: xp_ref.at[vib],
                   sdst=lambda vib, heb: ys.at[heb],
                   adjust=True, hist=compute_deg)
        plsc.subcore_barrier()

        col0 = s * MT
        if compute_deg:
            pltpu.sync_copy(cnt.at[pl.ds(col0, MT)], rbuf)

            @pl.loop(0, MT // L)
            def _(v):
                dv = rbuf[pl.ds(v * L, L)]
                rbuf[pl.ds(v * L, L)] = 1.0 / jnp.maximum(dv, 1.0)

            @pl.when(c == 0)
            def _():
                pltpu.sync_copy(rbuf, recip_out.at[pl.ds(col0, MT)])
        else:
            pltpu.sync_copy(recip_ref.at[pl.ds(col0, MT)], rbuf)

        for p in range(NPASS):
            pltpu.sync_copy(ys.at[pl.ds(col0 + p * TB, TB)], tbuf)

            @pl.loop(0, TB // L)
            def _(k):
                rv16 = rbuf[pl.ds(p * TB + k * L, L)]
                for i in range(L):
                    rv = jnp.broadcast_to(rv16[i], (L,))
                    for v in range(DH // L):
                        tbuf[k * L + i, pl.ds(v * L, L)] = (
                            tbuf[k * L + i, pl.ds(v * L, L)] * rv)

            pltpu.sync_copy(tbuf, ys.at[pl.ds(col0 + p * TB, TB)])
        plsc.subcore_barrier()

        _run_phase(gsrc=lambda vib, heb: ys.at[heb],
                   sdst=lambda vib, heb: xa.at[vib],
                   adjust=False, hist=False)
        plsc.subcore_barrier()

        for p in range(NPASS):
            pltpu.sync_copy(xa.at[pl.ds(s * MT + p * TB, TB)], tbuf)
            pltpu.sync_copy(tbuf, xa_out.at[c, pl.ds(s * MT + p * TB, TB)])

    run = pl.kernel(body, out_type=tuple(out_type), mesh=mesh,
                    scratch_types=scratch,
                    compiler_params=pltpu.CompilerParams(
                        use_tc_tiling_on_sc=False))
    if compute_deg:
        return run(xp_flat, vi1, he1)
    return run(xp_flat, vi1, he1, recip)[0]


def _tc_first(x, W, b):
    def body(x_ref, w_ref, b_ref, o_ref):
        h = jnp.dot(x_ref[...], w_ref[...],
                    preferred_element_type=jnp.float32) + b_ref[...]
        o_ref[0] = h[:, :DH]
        o_ref[1] = h[:, DH:]

    return pl.pallas_call(
        body,
        grid=(NP // BR,),
        in_specs=[pl.BlockSpec((BR, D), lambda i: (i, 0)),
                  pl.BlockSpec((D, D), lambda i: (0, 0)),
                  pl.BlockSpec((1, D), lambda i: (0, 0))],
        out_specs=pl.BlockSpec((NC, BR, DH), lambda i: (0, i, 0)),
        out_shape=jax.ShapeDtypeStruct((NC, NP, DH), jnp.float32),
    )(x, W, b.reshape(1, D))


def _tc_mid(xp, xa, W, b):
    def body(xp_ref, xa_ref, w_ref, b_ref, o_ref):
        h0 = jnp.maximum(xp_ref[0] + xa_ref[0], 0.0)
        h1 = jnp.maximum(xp_ref[1] + xa_ref[1], 0.0)
        hcat = jnp.concatenate([h0, h1], axis=1)
        h = jnp.dot(hcat, w_ref[...],
                    preferred_element_type=jnp.float32) + b_ref[...]
        o_ref[0] = h[:, :DH]
        o_ref[1] = h[:, DH:]

    return pl.pallas_call(
        body,
        grid=(NP // BR,),
        in_specs=[pl.BlockSpec((NC, BR, DH), lambda i: (0, i, 0)),
                  pl.BlockSpec((NC, BR, DH), lambda i: (0, i, 0)),
                  pl.BlockSpec((D, D), lambda i: (0, 0)),
                  pl.BlockSpec((1, D), lambda i: (0, 0))],
        out_specs=pl.BlockSpec((NC, BR, DH), lambda i: (0, i, 0)),
        out_shape=jax.ShapeDtypeStruct((NC, NP, DH), jnp.float32),
    )(xp, xa, W, b.reshape(1, D))


def _tc_final(xp, xa):
    def body(xp_ref, xa_ref, o_ref):
        h0 = jnp.maximum(xp_ref[0] + xa_ref[0], 0.0)
        h1 = jnp.maximum(xp_ref[1] + xa_ref[1], 0.0)
        o_ref[...] = jnp.concatenate([h0, h1], axis=1)

    return pl.pallas_call(
        body,
        grid=(NP // BR,),
        in_specs=[pl.BlockSpec((NC, BR, DH), lambda i: (0, i, 0)),
                  pl.BlockSpec((NC, BR, DH), lambda i: (0, i, 0))],
        out_specs=pl.BlockSpec((BR, D), lambda i: (i, 0)),
        out_shape=jax.ShapeDtypeStruct((NP, D), jnp.float32),
    )(xp, xa)


def kernel(x, vertex_idx, hedge_idx, W0, b0, W1, b1, W2, b2):
    pad = EP - E
    vpad = N + (jnp.arange(pad, dtype=jnp.int32) % (NP - N))
    hpad = M + (jnp.arange(pad, dtype=jnp.int32) % (MP - M))
    vi1 = jnp.concatenate([vertex_idx.astype(jnp.int32), vpad])
    he1 = jnp.concatenate([hedge_idx.astype(jnp.int32), hpad])
    xpad = jnp.pad(x, ((0, NP - N), (0, 0)))

    xp0 = _tc_first(xpad, W0, b0)
    xa0, recip = _sc_agg(xp0.reshape(NC * NP, DH), vi1, he1, None)
    xp1 = _tc_mid(xp0, xa0, W1, b1)
    xa1 = _sc_agg(xp1.reshape(NC * NP, DH), vi1, he1, recip)
    xp2 = _tc_mid(xp1, xa1, W2, b2)
    xa2 = _sc_agg(xp2.reshape(NC * NP, DH), vi1, he1, recip)
    return _tc_final(xp2, xa2)[:N]
